# baseline (speedup 1.0000x reference)
import jax
import jax.numpy as jnp
from jax.experimental import pallas as pl
from jax.experimental.pallas import tpu as pltpu

_STATIC_LEN = 8192
_HIDDEN = 1024
_CHUNK = 512
_NCHUNKS = _STATIC_LEN // _CHUNK
_NBUF = 12
_AHEAD = _NBUF - 2


def _copy_body(start_ref, src_ref, out_ref, *rest):
    bufs = rest[:_NBUF]
    lsems = rest[_NBUF:2 * _NBUF]
    ssems = rest[2 * _NBUF:3 * _NBUF]
    start = pl.multiple_of(start_ref[0], 8)

    def load(g):
        return pltpu.async_copy(
            src_ref.at[pl.ds(start + g * _CHUNK, _CHUNK), :],
            bufs[g % _NBUF], lsems[g % _NBUF])

    def store(g):
        return pltpu.async_copy(
            bufs[g % _NBUF],
            out_ref.at[pl.ds(g * _CHUNK, _CHUNK), :],
            ssems[g % _NBUF])

    loads = [None] * _NCHUNKS
    stores = [None] * _NCHUNKS
    for g in range(min(_AHEAD, _NCHUNKS)):
        loads[g] = load(g)
    for g in range(_NCHUNKS):
        idx = g + _AHEAD
        if idx < _NCHUNKS:
            if g >= 2:
                stores[g - 2].wait()
            loads[idx] = load(idx)
        loads[g].wait()
        stores[g] = store(g)
    for g in range(max(_NCHUNKS - _NBUF, 0), _NCHUNKS):
        stores[g].wait()


@jax.jit
def kernel(freqs, seq_len):
    src = freqs.reshape(_STATIC_LEN * 8, _HIDDEN)
    start = (jnp.asarray(seq_len, jnp.int32) - _STATIC_LEN).reshape(1)
    out = pl.pallas_call(
        _copy_body,
        out_shape=jax.ShapeDtypeStruct((_STATIC_LEN, _HIDDEN), jnp.float32),
        in_specs=[
            pl.BlockSpec(memory_space=pltpu.SMEM),
            pl.BlockSpec(memory_space=pl.ANY),
        ],
        out_specs=pl.BlockSpec(memory_space=pl.ANY),
        scratch_shapes=(
            [pltpu.VMEM((_CHUNK, _HIDDEN), jnp.float32)] * _NBUF
            + [pltpu.SemaphoreType.DMA] * (2 * _NBUF)
        ),
    )(start, src)
    return out.reshape(1, _STATIC_LEN, _HIDDEN)


# speedup vs baseline: 1.0052x; 1.0052x over previous
import jax
import jax.numpy as jnp
from jax.experimental import pallas as pl
from jax.experimental.pallas import tpu as pltpu

_STATIC_LEN = 8192
_HIDDEN = 1024
_CHUNK = 1024
_NCHUNKS = _STATIC_LEN // _CHUNK
_NBUF = 8
_AHEAD = _NBUF - 2


def _copy_body(start_ref, src_ref, out_ref, *rest):
    bufs = rest[:_NBUF]
    lsems = rest[_NBUF:2 * _NBUF]
    ssems = rest[2 * _NBUF:3 * _NBUF]
    start = pl.multiple_of(start_ref[0], 8)

    def load(g):
        return pltpu.async_copy(
            src_ref.at[pl.ds(start + g * _CHUNK, _CHUNK), :],
            bufs[g % _NBUF], lsems[g % _NBUF])

    def store(g):
        return pltpu.async_copy(
            bufs[g % _NBUF],
            out_ref.at[pl.ds(g * _CHUNK, _CHUNK), :],
            ssems[g % _NBUF])

    loads = [None] * _NCHUNKS
    stores = [None] * _NCHUNKS
    for g in range(min(_AHEAD, _NCHUNKS)):
        loads[g] = load(g)
    for g in range(_NCHUNKS):
        idx = g + _AHEAD
        if idx < _NCHUNKS:
            if g >= 2:
                stores[g - 2].wait()
            loads[idx] = load(idx)
        loads[g].wait()
        stores[g] = store(g)
    for g in range(max(_NCHUNKS - _NBUF, 0), _NCHUNKS):
        stores[g].wait()


@jax.jit
def kernel(freqs, seq_len):
    src = freqs.reshape(_STATIC_LEN * 8, _HIDDEN)
    start = (jnp.asarray(seq_len, jnp.int32) - _STATIC_LEN).reshape(1)
    out = pl.pallas_call(
        _copy_body,
        out_shape=jax.ShapeDtypeStruct((_STATIC_LEN, _HIDDEN), jnp.float32),
        in_specs=[
            pl.BlockSpec(memory_space=pltpu.SMEM),
            pl.BlockSpec(memory_space=pl.ANY),
        ],
        out_specs=pl.BlockSpec(memory_space=pl.ANY),
        scratch_shapes=(
            [pltpu.VMEM((_CHUNK, _HIDDEN), jnp.float32)] * _NBUF
            + [pltpu.SemaphoreType.DMA] * (2 * _NBUF)
        ),
    )(start, src)
    return out.reshape(1, _STATIC_LEN, _HIDDEN)
